# baseline (speedup 1.0000x reference)
import functools

import jax
import jax.numpy as jnp
from jax.experimental import pallas as pl
from jax.experimental.pallas import tpu as pltpu
from jax.experimental.pallas import tpu_sc as plsc

K = 65536
DIM = 128
IN_DIM = 256
B = 128
L = 50
C = 2
M_MOM = 0.999
THRESHOLD = 0.5
NUM_POS = 16
TEMP = 0.5
STRIDE = 10
EPS = 1e-8

S = (K + STRIDE - 1) // STRIDE
S_BLK = 256
S_PAD = ((S + S_BLK - 1) // S_BLK) * S_BLK
N_CHUNK = S_PAD // S_BLK
TAIL_ROWS = K - (N_CHUNK - 1) * S_BLK * STRIDE
TINY = 1e-20


def _compute_body(xl_ref, xq_ref, wq_ref, wk_ref, gs_ref, gl_ref, out_ref,
                  w_s, qh_s, nq_s, x0b_s, x1b_s, sel_s, ut_s, e0_s, e1_s,
                  cnt_s, acc_s, sbuf_s, sem_s):
    i = pl.program_id(0)

    @pl.when(i == 0)
    def _init():
        lam = jax.lax.broadcasted_iota(jnp.int32, (L, 2 * L), 0)
        jam = jax.lax.broadcasted_iota(jnp.int32, (L, 2 * L), 1)
        e0_s[...] = (jam == 2 * lam).astype(jnp.float32)
        e1_s[...] = (jam == 2 * lam + 1).astype(jnp.float32)
        selj = jax.lax.broadcasted_iota(jnp.int32, (S_BLK, S_BLK * STRIDE), 0)
        selr = jax.lax.broadcasted_iota(jnp.int32, (S_BLK, S_BLK * STRIDE), 1)
        sel_s[...] = (selr == STRIDE * selj).astype(jnp.float32)
        rowi = jax.lax.broadcasted_iota(jnp.int32, (S_BLK, S_BLK), 0)
        coli = jax.lax.broadcasted_iota(jnp.int32, (S_BLK, S_BLK), 1)
        ut_s[...] = (rowi < coli).astype(jnp.float32)

        w_s[...] = M_MOM * wk_ref[...] + (1.0 - M_MOM) * wq_ref[...]
        xq = xq_ref[...]
        nq = jnp.sqrt(jnp.sum(xq * xq, axis=1, keepdims=True))
        qh = xq / jnp.maximum(nq, EPS)
        qh_s[...] = qh
        nq_s[...] = jnp.sqrt(jnp.sum(qh * qh, axis=1, keepdims=True))
        xl = xl_ref[...]
        x0 = jax.lax.dot_general(xl, e0_s[...], (((1,), (1,)), ((), ())),
                                 preferred_element_type=jnp.float32)
        x1 = jax.lax.dot_general(xl, e1_s[...], (((1,), (1,)), ((), ())),
                                 preferred_element_type=jnp.float32)
        an = jnp.sqrt(x0 * x0 + x1 * x1)
        ran = 1.0 / jnp.maximum(an, TINY)
        x0b_s[...] = jnp.broadcast_to((x0 * ran)[:, :, None], (B, L, S_BLK))
        x1b_s[...] = jnp.broadcast_to((x1 * ran)[:, :, None], (B, L, S_BLK))
        cnt_s[...] = jnp.zeros_like(cnt_s)
        acc_s[...] = jnp.zeros_like(acc_s)

    need = jnp.min(cnt_s[...]) < NUM_POS

    @pl.when(need)
    def _heavy():
        rows = S_BLK * STRIDE

        @pl.when(i < N_CHUNK - 1)
        def _cp_full():
            cps = pltpu.make_async_copy(
                gs_ref.at[pl.ds(i * rows, rows), :], sbuf_s, sem_s)
            cps.start()
            cps.wait()

        @pl.when(i == N_CHUNK - 1)
        def _cp_tail():
            cps = pltpu.make_async_copy(
                gs_ref.at[pl.ds(i * rows, TAIL_ROWS), :],
                sbuf_s.at[pl.ds(0, TAIL_ROWS), :], sem_s)
            cps.start()
            cps.wait()

        _chunk_update(i, sbuf_s, gl_ref,
                      w_s, qh_s, nq_s, x0b_s, x1b_s, sel_s, ut_s, e0_s, e1_s,
                      cnt_s, acc_s)

    @pl.when(i == N_CHUNK - 1)
    def _fin():
        per = acc_s[...] / jnp.maximum(cnt_s[...], 1.0)
        out_ref[...] = jnp.sum(per).reshape(1, 1) / B


def _chunk_update(i, sbuf_s, gl_ref,
                  w_s, qh_s, nq_s, x0b_s, x1b_s, sel_s, ut_s, e0_s, e1_s,
                  cnt_s, acc_s):
    srows = jnp.dot(sel_s[...], sbuf_s[...],
                    preferred_element_type=jnp.float32)
    qs = jnp.dot(srows, w_s[...],
                 preferred_element_type=jnp.float32)
    nrm = jnp.sqrt(jnp.sum(qs * qs, axis=1, keepdims=True))
    qf = qs / jnp.maximum(nrm, EPS)
    ns = jnp.sqrt(jnp.sum(qf * qf, axis=1, keepdims=True))

    dot = jax.lax.dot_general(qh_s[...], qf, (((1,), (1,)), ((), ())),
                              preferred_element_type=jnp.float32)
    pn = jnp.maximum(nq_s[...] * ns.reshape(1, S_BLK), EPS)
    ps = dot / pn / TEMP
    loss_elem = -jnp.log(jax.nn.sigmoid(ps) + 1e-12)

    y = gl_ref[...]
    y0t = jax.lax.dot_general(e0_s[...], y, (((1,), (1,)), ((), ())),
                              preferred_element_type=jnp.float32)
    y1t = jax.lax.dot_general(e1_s[...], y, (((1,), (1,)), ((), ())),
                              preferred_element_type=jnp.float32)
    bn = jnp.sqrt(y0t * y0t + y1t * y1t)
    rbn = 1.0 / jnp.maximum(bn, TINY)
    y0t = y0t * rbn
    y1t = y1t * rbn
    num = x0b_s[...] * y0t[None, :, :] + x1b_s[...] * y1t[None, :, :]
    sim = jnp.sum(jnp.abs(num), axis=1) * (1.0 / L)

    lane = jax.lax.broadcasted_iota(jnp.int32, (B, S_BLK), 1)
    valid = (i * S_BLK + lane) < S
    mask = (sim >= THRESHOLD) & valid
    maskf = mask.astype(jnp.float32)
    excl = jnp.dot(maskf, ut_s[...], preferred_element_type=jnp.float32)
    w = mask & ((cnt_s[...] + excl) < NUM_POS)
    wf = w.astype(jnp.float32)
    acc_s[...] += jnp.sum(jnp.where(w, loss_elem, 0.0), axis=1, keepdims=True)
    cnt_s[...] += jnp.sum(wf, axis=1, keepdims=True)


@functools.partial(jax.jit, static_argnames=())
def _moco_loss(x_label2, x_q, W_q, W_k, g_samp, g_lab):
    out = pl.pallas_call(
        _compute_body,
        grid=(N_CHUNK,),
        in_specs=[
            pl.BlockSpec((B, 2 * L), lambda i: (0, 0)),
            pl.BlockSpec((B, DIM), lambda i: (0, 0)),
            pl.BlockSpec((IN_DIM, DIM), lambda i: (0, 0)),
            pl.BlockSpec((IN_DIM, DIM), lambda i: (0, 0)),
            pl.BlockSpec(memory_space=pltpu.MemorySpace.HBM),
            pl.BlockSpec((S_BLK, 2 * L), lambda i: (i, 0)),
        ],
        out_specs=pl.BlockSpec((1, 1), lambda i: (0, 0)),
        out_shape=jax.ShapeDtypeStruct((1, 1), jnp.float32),
        scratch_shapes=[
            pltpu.VMEM((IN_DIM, DIM), jnp.float32),
            pltpu.VMEM((B, DIM), jnp.float32),
            pltpu.VMEM((B, 1), jnp.float32),
            pltpu.VMEM((B, L, S_BLK), jnp.float32),
            pltpu.VMEM((B, L, S_BLK), jnp.float32),
            pltpu.VMEM((S_BLK, S_BLK * STRIDE), jnp.float32),
            pltpu.VMEM((S_BLK, S_BLK), jnp.float32),
            pltpu.VMEM((L, 2 * L), jnp.float32),
            pltpu.VMEM((L, 2 * L), jnp.float32),
            pltpu.VMEM((B, 1), jnp.float32),
            pltpu.VMEM((B, 1), jnp.float32),
            pltpu.VMEM((S_BLK * STRIDE, IN_DIM), jnp.float32),
            pltpu.SemaphoreType.DMA,
        ],
    )(x_label2, x_q, W_q, W_k, g_samp, g_lab)
    return out[0, 0]


GW = 128


def _sc_gather(sample2, idx2):
    mesh = plsc.VectorSubcoreMesh(core_axis_name="core",
                                  subcore_axis_name="subcore")

    @pl.kernel(
        out_type=jax.ShapeDtypeStruct((S_PAD, IN_DIM), jnp.float32),
        mesh=mesh,
    )
    def gather_kernel(s_hbm, i_hbm, os_hbm):
        def body(i_vmem, os_vmem):
            pltpu.sync_copy(s_hbm.at[i_vmem.at[0]], os_vmem)

        pltpu.emit_pipeline(
            body,
            grid=(S_PAD // GW,),
            in_specs=[pl.BlockSpec((1, GW), lambda i: (0, i))],
            out_specs=[pl.BlockSpec((GW, IN_DIM), lambda i: (i, 0))],
            core_axis_name=("core", "subcore"),
            dimension_semantics=(pltpu.PARALLEL,),
        )(i_hbm, os_hbm)

    return gather_kernel(sample2, idx2)


def kernel(x_q, x_label, sample_init, W_q, W_k, queue_labels):
    ql = queue_labels[::STRIDE]
    g_lab = jnp.pad(ql, ((0, S_PAD - S), (0, 0), (0, 0)),
                    mode="edge").reshape(S_PAD, L * C)
    x_label2 = x_label.reshape(B, L * C)
    return _moco_loss(x_label2, x_q, W_q, W_k, sample_init, g_lab)


# speedup vs baseline: 1.0123x; 1.0123x over previous
import functools

import jax
import jax.numpy as jnp
from jax.experimental import pallas as pl
from jax.experimental.pallas import tpu as pltpu
from jax.experimental.pallas import tpu_sc as plsc

K = 65536
DIM = 128
IN_DIM = 256
B = 128
L = 50
C = 2
M_MOM = 0.999
THRESHOLD = 0.5
NUM_POS = 16
TEMP = 0.5
STRIDE = 10
EPS = 1e-8

S = (K + STRIDE - 1) // STRIDE
S_BLK = 256
S_PAD = ((S + S_BLK - 1) // S_BLK) * S_BLK
N_CHUNK = S_PAD // S_BLK
TAIL_ROWS = K - (N_CHUNK - 1) * S_BLK * STRIDE
TINY = 1e-20


def _compute_body(xl_ref, xq_ref, wq_ref, wk_ref, gs_ref, gl_ref, out_ref,
                  w_s, qh_s, nq_s, x0b_s, x1b_s, sel_s, ut_s, e0_s, e1_s,
                  cnt_s, acc_s, sbuf_s, sem_s):
    i = pl.program_id(0)

    @pl.when(i == 0)
    def _init():
        lam = jax.lax.broadcasted_iota(jnp.int32, (L, 2 * L), 0)
        jam = jax.lax.broadcasted_iota(jnp.int32, (L, 2 * L), 1)
        e0_s[...] = (jam == 2 * lam).astype(jnp.float32)
        e1_s[...] = (jam == 2 * lam + 1).astype(jnp.float32)
        selj = jax.lax.broadcasted_iota(jnp.int32, (S_BLK, S_BLK * STRIDE), 0)
        selr = jax.lax.broadcasted_iota(jnp.int32, (S_BLK, S_BLK * STRIDE), 1)
        sel_s[...] = (selr == STRIDE * selj).astype(jnp.float32)
        rowi = jax.lax.broadcasted_iota(jnp.int32, (S_BLK, S_BLK), 0)
        coli = jax.lax.broadcasted_iota(jnp.int32, (S_BLK, S_BLK), 1)
        ut_s[...] = (rowi < coli).astype(jnp.float32)

        w_s[...] = M_MOM * wk_ref[...] + (1.0 - M_MOM) * wq_ref[...]
        xq = xq_ref[...]
        nq = jnp.sqrt(jnp.sum(xq * xq, axis=1, keepdims=True))
        qh = xq / jnp.maximum(nq, EPS)
        qh_s[...] = qh
        nq_s[...] = jnp.sqrt(jnp.sum(qh * qh, axis=1, keepdims=True))
        xl = xl_ref[...]
        x0 = jax.lax.dot_general(xl, e0_s[...], (((1,), (1,)), ((), ())),
                                 preferred_element_type=jnp.float32)
        x1 = jax.lax.dot_general(xl, e1_s[...], (((1,), (1,)), ((), ())),
                                 preferred_element_type=jnp.float32)
        an = jnp.sqrt(x0 * x0 + x1 * x1)
        ran = 1.0 / jnp.maximum(an, TINY)
        x0b_s[...] = jnp.broadcast_to((x0 * ran)[:, :, None], (B, L, S_BLK))
        x1b_s[...] = jnp.broadcast_to((x1 * ran)[:, :, None], (B, L, S_BLK))
        cnt_s[...] = jnp.zeros_like(cnt_s)
        acc_s[...] = jnp.zeros_like(acc_s)

    need = jnp.min(cnt_s[...]) < NUM_POS

    @pl.when(need)
    def _heavy():
        rows = S_BLK * STRIDE

        @pl.when(i < N_CHUNK - 1)
        def _cp_full():
            cps = pltpu.make_async_copy(
                gs_ref.at[pl.ds(i * rows, rows), :], sbuf_s, sem_s)
            cps.start()
            cps.wait()

        @pl.when(i == N_CHUNK - 1)
        def _cp_tail():
            cps = pltpu.make_async_copy(
                gs_ref.at[pl.ds(i * rows, TAIL_ROWS), :],
                sbuf_s.at[pl.ds(0, TAIL_ROWS), :], sem_s)
            cps.start()
            cps.wait()

        _chunk_update(i, sbuf_s, gl_ref,
                      w_s, qh_s, nq_s, x0b_s, x1b_s, sel_s, ut_s, e0_s, e1_s,
                      cnt_s, acc_s)

    @pl.when(i == N_CHUNK - 1)
    def _fin():
        per = acc_s[...] / jnp.maximum(cnt_s[...], 1.0)
        out_ref[...] = jnp.sum(per).reshape(1, 1) / B


def _chunk_update(i, sbuf_s, gl_ref,
                  w_s, qh_s, nq_s, x0b_s, x1b_s, sel_s, ut_s, e0_s, e1_s,
                  cnt_s, acc_s):
    srows = jnp.dot(sel_s[...], sbuf_s[...],
                    preferred_element_type=jnp.float32)
    qs = jnp.dot(srows, w_s[...],
                 preferred_element_type=jnp.float32)
    nrm = jnp.sqrt(jnp.sum(qs * qs, axis=1, keepdims=True))
    qf = qs / jnp.maximum(nrm, EPS)
    ns = jnp.sqrt(jnp.sum(qf * qf, axis=1, keepdims=True))

    dot = jax.lax.dot_general(qh_s[...], qf, (((1,), (1,)), ((), ())),
                              preferred_element_type=jnp.float32)
    pn = jnp.maximum(nq_s[...] * ns.reshape(1, S_BLK), EPS)
    ps = dot / pn / TEMP
    loss_elem = -jnp.log(jax.nn.sigmoid(ps) + 1e-12)

    y0t = gl_ref[:L, :]
    y1t = gl_ref[L:, :]
    bn = jnp.sqrt(y0t * y0t + y1t * y1t)
    rbn = 1.0 / jnp.maximum(bn, TINY)
    y0t = y0t * rbn
    y1t = y1t * rbn
    num = x0b_s[...] * y0t[None, :, :] + x1b_s[...] * y1t[None, :, :]
    sim = jnp.sum(jnp.abs(num), axis=1) * (1.0 / L)

    lane = jax.lax.broadcasted_iota(jnp.int32, (B, S_BLK), 1)
    valid = (i * S_BLK + lane) < S
    mask = (sim >= THRESHOLD) & valid
    maskf = mask.astype(jnp.float32)
    excl = jnp.dot(maskf, ut_s[...], preferred_element_type=jnp.float32)
    w = mask & ((cnt_s[...] + excl) < NUM_POS)
    wf = w.astype(jnp.float32)
    acc_s[...] += jnp.sum(jnp.where(w, loss_elem, 0.0), axis=1, keepdims=True)
    cnt_s[...] += jnp.sum(wf, axis=1, keepdims=True)


@functools.partial(jax.jit, static_argnames=())
def _moco_loss(x_label2, x_q, W_q, W_k, g_samp, g_lab):
    out = pl.pallas_call(
        _compute_body,
        grid=(N_CHUNK,),
        in_specs=[
            pl.BlockSpec((B, 2 * L), lambda i: (0, 0)),
            pl.BlockSpec((B, DIM), lambda i: (0, 0)),
            pl.BlockSpec((IN_DIM, DIM), lambda i: (0, 0)),
            pl.BlockSpec((IN_DIM, DIM), lambda i: (0, 0)),
            pl.BlockSpec(memory_space=pltpu.MemorySpace.HBM),
            pl.BlockSpec((2 * L, S_BLK), lambda i: (0, i)),
        ],
        out_specs=pl.BlockSpec((1, 1), lambda i: (0, 0)),
        out_shape=jax.ShapeDtypeStruct((1, 1), jnp.float32),
        scratch_shapes=[
            pltpu.VMEM((IN_DIM, DIM), jnp.float32),
            pltpu.VMEM((B, DIM), jnp.float32),
            pltpu.VMEM((B, 1), jnp.float32),
            pltpu.VMEM((B, L, S_BLK), jnp.float32),
            pltpu.VMEM((B, L, S_BLK), jnp.float32),
            pltpu.VMEM((S_BLK, S_BLK * STRIDE), jnp.float32),
            pltpu.VMEM((S_BLK, S_BLK), jnp.float32),
            pltpu.VMEM((L, 2 * L), jnp.float32),
            pltpu.VMEM((L, 2 * L), jnp.float32),
            pltpu.VMEM((B, 1), jnp.float32),
            pltpu.VMEM((B, 1), jnp.float32),
            pltpu.VMEM((S_BLK * STRIDE, IN_DIM), jnp.float32),
            pltpu.SemaphoreType.DMA,
        ],
    )(x_label2, x_q, W_q, W_k, g_samp, g_lab)
    return out[0, 0]


GW = 128


def _sc_gather(sample2, idx2):
    mesh = plsc.VectorSubcoreMesh(core_axis_name="core",
                                  subcore_axis_name="subcore")

    @pl.kernel(
        out_type=jax.ShapeDtypeStruct((S_PAD, IN_DIM), jnp.float32),
        mesh=mesh,
    )
    def gather_kernel(s_hbm, i_hbm, os_hbm):
        def body(i_vmem, os_vmem):
            pltpu.sync_copy(s_hbm.at[i_vmem.at[0]], os_vmem)

        pltpu.emit_pipeline(
            body,
            grid=(S_PAD // GW,),
            in_specs=[pl.BlockSpec((1, GW), lambda i: (0, i))],
            out_specs=[pl.BlockSpec((GW, IN_DIM), lambda i: (i, 0))],
            core_axis_name=("core", "subcore"),
            dimension_semantics=(pltpu.PARALLEL,),
        )(i_hbm, os_hbm)

    return gather_kernel(sample2, idx2)


def kernel(x_q, x_label, sample_init, W_q, W_k, queue_labels):
    ql = queue_labels[::STRIDE]
    glt = jnp.concatenate([ql[:, :, 0].T, ql[:, :, 1].T], axis=0)
    g_lab = jnp.pad(glt, ((0, 0), (0, S_PAD - S)), constant_values=1.0)
    x_label2 = x_label.reshape(B, L * C)
    return _moco_loss(x_label2, x_q, W_q, W_k, sample_init, g_lab)


# speedup vs baseline: 1.6701x; 1.6498x over previous
import functools

import jax
import jax.numpy as jnp
from jax.experimental import pallas as pl
from jax.experimental.pallas import tpu as pltpu
from jax.experimental.pallas import tpu_sc as plsc

K = 65536
DIM = 128
IN_DIM = 256
B = 128
L = 50
C = 2
M_MOM = 0.999
THRESHOLD = 0.5
NUM_POS = 16
TEMP = 0.5
STRIDE = 10
EPS = 1e-8

S = (K + STRIDE - 1) // STRIDE
S_BLK = 256
S_PAD = ((S + S_BLK - 1) // S_BLK) * S_BLK
N_CHUNK = S_PAD // S_BLK
TAIL_ROWS = K - (N_CHUNK - 1) * S_BLK * STRIDE
TINY = 1e-20


def _compute_body(xl_ref, xq_ref, wq_ref, wk_ref, gs_ref, gl_ref, out_ref,
                  w_s, qh_s, nq_s, x0b_s, x1b_s, sel_s, ut_s, e0_s, e1_s,
                  cnt_s, acc_s, sbuf_s, sem_s):
    i = pl.program_id(0)

    @pl.when(i == 0)
    def _init():
        lam = jax.lax.broadcasted_iota(jnp.int32, (L, 2 * L), 0)
        jam = jax.lax.broadcasted_iota(jnp.int32, (L, 2 * L), 1)
        e0_s[...] = (jam == 2 * lam).astype(jnp.float32)
        e1_s[...] = (jam == 2 * lam + 1).astype(jnp.float32)
        selj = jax.lax.broadcasted_iota(jnp.int32, (S_BLK, S_BLK * STRIDE), 0)
        selr = jax.lax.broadcasted_iota(jnp.int32, (S_BLK, S_BLK * STRIDE), 1)
        sel_s[...] = (selr == STRIDE * selj).astype(jnp.float32)
        rowi = jax.lax.broadcasted_iota(jnp.int32, (S_BLK, S_BLK), 0)
        coli = jax.lax.broadcasted_iota(jnp.int32, (S_BLK, S_BLK), 1)
        ut_s[...] = (rowi < coli).astype(jnp.float32)

        w_s[...] = M_MOM * wk_ref[...] + (1.0 - M_MOM) * wq_ref[...]
        xq = xq_ref[...]
        nq = jnp.sqrt(jnp.sum(xq * xq, axis=1, keepdims=True))
        qh = xq / jnp.maximum(nq, EPS)
        qh_s[...] = qh
        nq_s[...] = jnp.sqrt(jnp.sum(qh * qh, axis=1, keepdims=True))
        xl = xl_ref[...]
        x0 = jax.lax.dot_general(xl, e0_s[...], (((1,), (1,)), ((), ())),
                                 preferred_element_type=jnp.float32)
        x1 = jax.lax.dot_general(xl, e1_s[...], (((1,), (1,)), ((), ())),
                                 preferred_element_type=jnp.float32)
        an = jnp.sqrt(x0 * x0 + x1 * x1)
        ran = 1.0 / jnp.maximum(an, TINY)
        x0b_s[...] = jnp.broadcast_to((x0 * ran)[:, :, None], (B, L, S_BLK))
        x1b_s[...] = jnp.broadcast_to((x1 * ran)[:, :, None], (B, L, S_BLK))
        cnt_s[...] = jnp.zeros_like(cnt_s)
        acc_s[...] = jnp.zeros_like(acc_s)

    need = jnp.min(cnt_s[...]) < NUM_POS

    @pl.when(need)
    def _heavy():
        rows = S_BLK * STRIDE

        @pl.when(i < N_CHUNK - 1)
        def _cp_full():
            cps = pltpu.make_async_copy(
                gs_ref.at[pl.ds(i * rows, rows), :], sbuf_s, sem_s)
            cps.start()
            cps.wait()

        @pl.when(i == N_CHUNK - 1)
        def _cp_tail():
            cps = pltpu.make_async_copy(
                gs_ref.at[pl.ds(i * rows, TAIL_ROWS), :],
                sbuf_s.at[pl.ds(0, TAIL_ROWS), :], sem_s)
            cps.start()
            cps.wait()

        _chunk_update(i, sbuf_s, gl_ref,
                      w_s, qh_s, nq_s, x0b_s, x1b_s, sel_s, ut_s, e0_s, e1_s,
                      cnt_s, acc_s)

    @pl.when(i == N_CHUNK - 1)
    def _fin():
        per = acc_s[...] / jnp.maximum(cnt_s[...], 1.0)
        out_ref[...] = jnp.sum(per).reshape(1, 1) / B


def _chunk_update(i, sbuf_s, gl_ref,
                  w_s, qh_s, nq_s, x0b_s, x1b_s, sel_s, ut_s, e0_s, e1_s,
                  cnt_s, acc_s):
    srows = jnp.dot(sel_s[...], sbuf_s[...],
                    preferred_element_type=jnp.float32)
    qs = jnp.dot(srows, w_s[...],
                 preferred_element_type=jnp.float32)
    nrm = jnp.sqrt(jnp.sum(qs * qs, axis=1, keepdims=True))
    qf = qs / jnp.maximum(nrm, EPS)
    ns = jnp.sqrt(jnp.sum(qf * qf, axis=1, keepdims=True))

    dot = jax.lax.dot_general(qh_s[...], qf, (((1,), (1,)), ((), ())),
                              preferred_element_type=jnp.float32)
    pn = jnp.maximum(nq_s[...] * ns.reshape(1, S_BLK), EPS)
    ps = dot / pn / TEMP
    loss_elem = -jnp.log(jax.nn.sigmoid(ps) + 1e-12)

    y0t = gl_ref[:L, :]
    y1t = gl_ref[L:, :]
    bn = jnp.sqrt(y0t * y0t + y1t * y1t)
    rbn = 1.0 / jnp.maximum(bn, TINY)
    y0t = y0t * rbn
    y1t = y1t * rbn
    num = x0b_s[...] * y0t[None, :, :] + x1b_s[...] * y1t[None, :, :]
    sim = jnp.sum(jnp.abs(num), axis=1) * (1.0 / L)

    lane = jax.lax.broadcasted_iota(jnp.int32, (B, S_BLK), 1)
    valid = (i * S_BLK + lane) < S
    mask = (sim >= THRESHOLD) & valid
    maskf = mask.astype(jnp.float32)
    excl = jnp.dot(maskf, ut_s[...], preferred_element_type=jnp.float32)
    w = mask & ((cnt_s[...] + excl) < NUM_POS)
    wf = w.astype(jnp.float32)
    acc_s[...] += jnp.sum(jnp.where(w, loss_elem, 0.0), axis=1, keepdims=True)
    cnt_s[...] += jnp.sum(wf, axis=1, keepdims=True)


@functools.partial(jax.jit, static_argnames=())
def _moco_loss(x_label2, x_q, W_q, W_k, g_samp, g_lab):
    out = pl.pallas_call(
        _compute_body,
        grid=(N_CHUNK,),
        in_specs=[
            pl.BlockSpec((B, 2 * L), lambda i: (0, 0)),
            pl.BlockSpec((B, DIM), lambda i: (0, 0)),
            pl.BlockSpec((IN_DIM, DIM), lambda i: (0, 0)),
            pl.BlockSpec((IN_DIM, DIM), lambda i: (0, 0)),
            pl.BlockSpec(memory_space=pltpu.MemorySpace.HBM),
            pl.BlockSpec((2 * L, S_BLK), lambda i: (0, i)),
        ],
        out_specs=pl.BlockSpec((1, 1), lambda i: (0, 0)),
        out_shape=jax.ShapeDtypeStruct((1, 1), jnp.float32),
        scratch_shapes=[
            pltpu.VMEM((IN_DIM, DIM), jnp.float32),
            pltpu.VMEM((B, DIM), jnp.float32),
            pltpu.VMEM((B, 1), jnp.float32),
            pltpu.VMEM((B, L, S_BLK), jnp.float32),
            pltpu.VMEM((B, L, S_BLK), jnp.float32),
            pltpu.VMEM((S_BLK, S_BLK * STRIDE), jnp.float32),
            pltpu.VMEM((S_BLK, S_BLK), jnp.float32),
            pltpu.VMEM((L, 2 * L), jnp.float32),
            pltpu.VMEM((L, 2 * L), jnp.float32),
            pltpu.VMEM((B, 1), jnp.float32),
            pltpu.VMEM((B, 1), jnp.float32),
            pltpu.VMEM((S_BLK * STRIDE, IN_DIM), jnp.float32),
            pltpu.SemaphoreType.DMA,
        ],
    )(x_label2, x_q, W_q, W_k, g_samp, g_lab)
    return out[0, 0]


GW = 128


def _sc_gather(sample2, idx2):
    mesh = plsc.VectorSubcoreMesh(core_axis_name="core",
                                  subcore_axis_name="subcore")

    @pl.kernel(
        out_type=jax.ShapeDtypeStruct((S_PAD, IN_DIM), jnp.float32),
        mesh=mesh,
    )
    def gather_kernel(s_hbm, i_hbm, os_hbm):
        def body(i_vmem, os_vmem):
            pltpu.sync_copy(s_hbm.at[i_vmem.at[0]], os_vmem)

        pltpu.emit_pipeline(
            body,
            grid=(S_PAD // GW,),
            in_specs=[pl.BlockSpec((1, GW), lambda i: (0, i))],
            out_specs=[pl.BlockSpec((GW, IN_DIM), lambda i: (i, 0))],
            core_axis_name=("core", "subcore"),
            dimension_semantics=(pltpu.PARALLEL,),
        )(i_hbm, os_hbm)

    return gather_kernel(sample2, idx2)


def kernel(x_q, x_label, sample_init, W_q, W_k, queue_labels):
    idx = jnp.minimum(jnp.arange(S_PAD, dtype=jnp.int32) * STRIDE, (S - 1) * STRIDE)
    ql = jnp.take(queue_labels, idx, axis=0)
    g_lab = jnp.concatenate([ql[:, :, 0].T, ql[:, :, 1].T], axis=0)
    x_label2 = x_label.reshape(B, L * C)
    return _moco_loss(x_label2, x_q, W_q, W_k, sample_init, g_lab)
